# baseline (speedup 1.0000x reference)
import functools

import jax
import jax.numpy as jnp
from jax import lax
from jax.experimental import pallas as pl
from jax.experimental.pallas import tpu as pltpu
from jax.experimental.pallas import tpu_sc as plsc

_NC = 2
_NS = 16
_NW = _NC * _NS
_L = 16


def _pool_sc_body(x_hbm, out_hbm, buf0, buf1, prow, sem0, sem1, *,
                  row_words, rows_per_tile, chunks_per_row, chunk):
    wid = lax.axis_index("s") * _NC + lax.axis_index("c")
    nchunk = rows_per_tile * chunks_per_row
    base = pl.multiple_of(wid * rows_per_tile * row_words, 8)
    bufs = (buf0, buf1)
    sems = (sem0, sem1)

    pltpu.async_copy(x_hbm.at[pl.ds(base, chunk)], buf0, sem0)

    @pl.loop(0, rows_per_tile, step=2)
    def _rows(g):
        for rr in range(2):
            r = g + rr
            accs = [jnp.zeros((_L,), jnp.float32)] * 8
            for c in range(chunks_per_row):
                parity = (3 * rr + c) % 2
                buf = bufs[parity]
                gi = r * chunks_per_row + c
                nxt = gi + 1

                @pl.when(nxt < nchunk)
                def _start_next():
                    off = pl.multiple_of(base + nxt * chunk, 8)
                    pltpu.async_copy(x_hbm.at[pl.ds(off, chunk)],
                                     bufs[1 - parity], sems[1 - parity])

                pltpu.make_async_copy(x_hbm.at[pl.ds(base, chunk)], buf,
                                      sems[parity]).wait()

                @pl.loop(0, chunk // (8 * _L), init_carry=tuple(accs))
                def _acc(i, acc):
                    o = pl.multiple_of(i * (8 * _L), 8 * _L)
                    return tuple(a + buf[pl.ds(o + _L * k, _L)]
                                 for k, a in enumerate(acc))

                accs = list(_acc)

            a = accs
            v = ((a[0] + a[1]) + (a[2] + a[3])) + \
                ((a[4] + a[5]) + (a[6] + a[7]))
            prow[pl.ds(pl.multiple_of(r * _L, _L), _L)] = v

    pltpu.sync_copy(prow, out_hbm.at[pl.ds(wid * rows_per_tile * _L,
                                           rows_per_tile * _L)])


def _proj_body(p_ref, m_ref, o_ref, *, inv_n):
    s = jnp.sum(p_ref[...], axis=1, keepdims=True) * inv_n
    o_ref[...] = jax.lax.dot_general(
        m_ref[...],
        s,
        (((1,), (0,)), ((), ())),
        preferred_element_type=jnp.float32,
    )


def kernel(x, W):
    B, C, H, Wd = x.shape
    N = H * Wd
    E = W.shape[0]
    R = B * C

    rows_per_tile = R // _NW
    chunks_per_row = 3
    chunk = N // chunks_per_row

    xf = x.reshape(R * N)

    mesh = plsc.VectorSubcoreMesh(core_axis_name="c", subcore_axis_name="s")
    rowsums = pl.kernel(
        functools.partial(
            _pool_sc_body,
            row_words=N,
            rows_per_tile=rows_per_tile,
            chunks_per_row=chunks_per_row,
            chunk=chunk,
        ),
        out_type=jax.ShapeDtypeStruct((R * _L,), jnp.float32),
        mesh=mesh,
        scratch_types=[
            pltpu.VMEM((chunk,), jnp.float32),
            pltpu.VMEM((chunk,), jnp.float32),
            pltpu.VMEM((rows_per_tile * _L,), jnp.float32),
            pltpu.SemaphoreType.DMA,
            pltpu.SemaphoreType.DMA,
        ],
    )(xf)

    prows = rowsums.reshape(R, _L)

    M = (jnp.eye(B, dtype=jnp.float32)[:, None, :, None]
         * W[None, :, None, :]).reshape(B * E, R)

    logits_flat = pl.pallas_call(
        functools.partial(_proj_body, inv_n=1.0 / N),
        in_specs=[
            pl.BlockSpec((R, _L), lambda: (0, 0)),
            pl.BlockSpec((B * E, R), lambda: (0, 0)),
        ],
        out_specs=pl.BlockSpec((B * E, 1), lambda: (0, 0)),
        out_shape=jax.ShapeDtypeStruct((B * E, 1), jnp.float32),
    )(prows, M)

    return logits_flat.reshape(B, E)


# speedup vs baseline: 1.0320x; 1.0320x over previous
import functools

import jax
import jax.numpy as jnp
from jax import lax
from jax.experimental import pallas as pl
from jax.experimental.pallas import tpu as pltpu
from jax.experimental.pallas import tpu_sc as plsc

_NC = 2
_NS = 16
_NW = _NC * _NS
_L = 16


_NBUF = 4
_AHEAD = 3


def _pool_sc_body(x_hbm, out_hbm, *refs, row_words, rows_per_tile,
                  chunks_per_row, chunk):
    bufs = refs[:_NBUF]
    sems = refs[_NBUF + 1:]
    prow = refs[_NBUF]
    wid = lax.axis_index("s") * _NC + lax.axis_index("c")
    nchunk = rows_per_tile * chunks_per_row
    base = pl.multiple_of(wid * rows_per_tile * row_words, 8)

    def start(gi, slot):
        off = pl.multiple_of(base + gi * chunk, 8)
        pltpu.async_copy(x_hbm.at[pl.ds(off, chunk)], bufs[slot], sems[slot])

    for g in range(_AHEAD):
        start(g, g)

    @pl.loop(0, rows_per_tile)
    def _rows(r):
        accs = [jnp.zeros((_L,), jnp.float32)] * 8
        for c in range(chunks_per_row):
            slot = c % _NBUF
            gi = r * chunks_per_row + c
            nxt = gi + _AHEAD

            @pl.when(nxt < nchunk)
            def _start_next():
                start(nxt, (c + _AHEAD) % _NBUF)

            buf = bufs[slot]
            pltpu.make_async_copy(x_hbm.at[pl.ds(base, chunk)], buf,
                                  sems[slot]).wait()

            @pl.loop(0, chunk // (8 * _L), init_carry=tuple(accs))
            def _acc(i, acc):
                o = pl.multiple_of(i * (8 * _L), 8 * _L)
                return tuple(a + buf[pl.ds(o + _L * k, _L)]
                             for k, a in enumerate(acc))

            accs = list(_acc)

        a = accs
        v = ((a[0] + a[1]) + (a[2] + a[3])) + \
            ((a[4] + a[5]) + (a[6] + a[7]))
        prow[pl.ds(pl.multiple_of(r * _L, _L), _L)] = v

    pltpu.sync_copy(prow, out_hbm.at[pl.ds(wid * rows_per_tile * _L,
                                           rows_per_tile * _L)])


def _proj_body(p_ref, m_ref, o_ref, *, inv_n):
    s = jnp.sum(p_ref[...], axis=1, keepdims=True) * inv_n
    o_ref[...] = jax.lax.dot_general(
        m_ref[...],
        s,
        (((1,), (0,)), ((), ())),
        preferred_element_type=jnp.float32,
    )


def kernel(x, W):
    B, C, H, Wd = x.shape
    N = H * Wd
    E = W.shape[0]
    R = B * C

    rows_per_tile = R // _NW
    chunks_per_row = 8
    chunk = N // chunks_per_row

    xf = x.reshape(R * N)

    mesh = plsc.VectorSubcoreMesh(core_axis_name="c", subcore_axis_name="s")
    rowsums = pl.kernel(
        functools.partial(
            _pool_sc_body,
            row_words=N,
            rows_per_tile=rows_per_tile,
            chunks_per_row=chunks_per_row,
            chunk=chunk,
        ),
        out_type=jax.ShapeDtypeStruct((R * _L,), jnp.float32),
        mesh=mesh,
        scratch_types=(
            [pltpu.VMEM((chunk,), jnp.float32)] * _NBUF
            + [pltpu.VMEM((rows_per_tile * _L,), jnp.float32)]
            + [pltpu.SemaphoreType.DMA] * _NBUF
        ),
    )(xf)

    prows = rowsums.reshape(R, _L)

    M = (jnp.eye(B, dtype=jnp.float32)[:, None, :, None]
         * W[None, :, None, :]).reshape(B * E, R)

    logits_flat = pl.pallas_call(
        functools.partial(_proj_body, inv_n=1.0 / N),
        in_specs=[
            pl.BlockSpec((R, _L), lambda: (0, 0)),
            pl.BlockSpec((B * E, R), lambda: (0, 0)),
        ],
        out_specs=pl.BlockSpec((B * E, 1), lambda: (0, 0)),
        out_shape=jax.ShapeDtypeStruct((B * E, 1), jnp.float32),
    )(prows, M)

    return logits_flat.reshape(B, E)


# speedup vs baseline: 1.3216x; 1.2806x over previous
import functools

import jax
import jax.numpy as jnp
from jax import lax
from jax.experimental import pallas as pl
from jax.experimental.pallas import tpu as pltpu

_NBUF = 8


def _tc_pool_body(x_hbm, o_ref, vmem, sem, *, nslab, slab_rows):
    def start(si):
        slot = lax.rem(si, _NBUF)
        pltpu.make_async_copy(
            x_hbm.at[pl.ds(si * slab_rows, slab_rows), :],
            vmem.at[slot],
            sem.at[slot],
        ).start()

    for s in range(_NBUF - 1):
        start(s)

    def step(si, _):
        slot = lax.rem(si, _NBUF)
        nxt = si + _NBUF - 1

        @pl.when(nxt < nslab)
        def _():
            start(nxt)

        pltpu.make_async_copy(
            x_hbm.at[pl.ds(si * slab_rows, slab_rows), :],
            vmem.at[slot],
            sem.at[slot],
        ).wait()
        o_ref[pl.ds(si * slab_rows, slab_rows)] = jnp.sum(
            vmem[slot], axis=1, keepdims=True
        )
        return 0

    lax.fori_loop(0, nslab, step, 0)


def _proj_body(p_ref, m_ref, o_ref, *, inv_n):
    s = p_ref[...] * inv_n
    o_ref[...] = jax.lax.dot_general(
        m_ref[...],
        s,
        (((1,), (0,)), ((), ())),
        preferred_element_type=jnp.float32,
    )


def kernel(x, W):
    B, C, H, Wd = x.shape
    N = H * Wd
    E = W.shape[0]
    R = B * C

    slab_rows = 8
    nslab = R // slab_rows

    xf = x.reshape(R, N)

    rowsums = pl.pallas_call(
        functools.partial(_tc_pool_body, nslab=nslab, slab_rows=slab_rows),
        in_specs=[pl.BlockSpec(memory_space=pl.ANY)],
        out_specs=pl.BlockSpec(memory_space=pltpu.MemorySpace.VMEM),
        out_shape=jax.ShapeDtypeStruct((R, 1), jnp.float32),
        scratch_shapes=[
            pltpu.VMEM((_NBUF, slab_rows, N), jnp.float32),
            pltpu.SemaphoreType.DMA((_NBUF,)),
        ],
        compiler_params=pltpu.CompilerParams(
            vmem_limit_bytes=100 * 1024 * 1024,
        ),
    )(xf)

    M = (jnp.eye(B, dtype=jnp.float32)[:, None, :, None]
         * W[None, :, None, :]).reshape(B * E, R)

    logits_flat = pl.pallas_call(
        functools.partial(_proj_body, inv_n=1.0 / N),
        in_specs=[
            pl.BlockSpec((R, 1), lambda: (0, 0)),
            pl.BlockSpec((B * E, R), lambda: (0, 0)),
        ],
        out_specs=pl.BlockSpec((B * E, 1), lambda: (0, 0)),
        out_shape=jax.ShapeDtypeStruct((B * E, 1), jnp.float32),
    )(rowsums, M)

    return logits_flat.reshape(B, E)
